# baseline (speedup 1.0000x reference)
import functools

import jax
import jax.numpy as jnp
from jax import lax
from jax.experimental import pallas as pl
from jax.experimental.pallas import tpu as pltpu
from jax.experimental.pallas import tpu_sc as plsc

N = 10000
E = 320000
NC = 2
NS = 16
NW = NC * NS
EPW = E // NW
CH = 80
RPT = N // NS
BE = 2560
GRID = E // BE
EPS = 1e-5

_mesh = functools.partial(
    plsc.VectorSubcoreMesh,
    core_axis_name="c", subcore_axis_name="s", num_cores=NC, num_subcores=NS)

_SC_PARAMS = pltpu.CompilerParams(use_tc_tiling_on_sc=False)


@functools.cache
def _make_gather(w):
  e = w - 8

  @functools.partial(
      pl.kernel,
      out_type=(jax.ShapeDtypeStruct((E, e), jnp.float32),
                jax.ShapeDtypeStruct((E, e), jnp.float32),
                jax.ShapeDtypeStruct((E, 8), jnp.float32),
                jax.ShapeDtypeStruct((E, 8), jnp.float32),
                jax.ShapeDtypeStruct((8,), jnp.float32)),
      mesh=_mesh(),
      scratch_types=[pltpu.VMEM((EPW,), jnp.int32),
                     pltpu.VMEM((EPW,), jnp.int32),
                     pltpu.VMEM((CH, w), jnp.float32),
                     pltpu.VMEM((CH, w), jnp.float32),
                     pltpu.VMEM((CH, w), jnp.float32),
                     pltpu.VMEM((CH, w), jnp.float32),
                     pltpu.VMEM((8,), jnp.float32),
                     pltpu.SemaphoreType.DMA,
                     pltpu.SemaphoreType.DMA,
                     pltpu.SemaphoreType.DMA],
      compiler_params=_SC_PARAMS)
  def gather_k(t_h, src_h, dst_h, tok_h, hI_h, hJ_h, pI_h, pJ_h, tok_o,
               idxd, idxs, bg0, bf0, bg1, bf1, tokb, sem0, sem1, semw):
    wid = lax.axis_index("s") * NC + lax.axis_index("c")
    base0 = wid * EPW

    @pl.when(wid == 0)
    def _():
      pltpu.sync_copy(tok_h, tokb)
      pltpu.sync_copy(tokb, tok_o)
    pltpu.sync_copy(dst_h.at[pl.ds(base0, EPW)], idxd)
    pltpu.sync_copy(src_h.at[pl.ds(base0, EPW)], idxs)
    npairs = EPW // (2 * CH)

    def fire(off, bg, bf, sem):
      cg = pltpu.async_copy(t_h.at[idxd.at[pl.ds(off, CH)]], bg, sem)
      cf = pltpu.async_copy(t_h.at[idxs.at[pl.ds(off, CH)]], bf, sem)
      return cg, cf

    def drain(off, bg, bf, cg, cf):
      cg.wait()
      cf.wait()
      sl = pl.ds(base0 + off, CH)
      wg = pltpu.async_copy(bg.at[:, pl.ds(0, e)], hI_h.at[sl], semw)
      wf = pltpu.async_copy(bf.at[:, pl.ds(0, e)], hJ_h.at[sl], semw)
      wp = pltpu.async_copy(bg.at[:, pl.ds(e, 8)], pI_h.at[sl], semw)
      wq = pltpu.async_copy(bf.at[:, pl.ds(e, 8)], pJ_h.at[sl], semw)
      return wg, wf, wp, wq

    def body(pi, carry):
      off0 = 2 * pi * CH
      off1 = off0 + CH
      cg0, cf0 = fire(off0, bg0, bf0, sem0)
      cg1, cf1 = fire(off1, bg1, bf1, sem1)
      ws0 = drain(off0, bg0, bf0, cg0, cf0)
      ws1 = drain(off1, bg1, bf1, cg1, cf1)
      for c in ws0 + ws1:
        c.wait()
      return carry

    lax.fori_loop(0, npairs, body, 0)
    for off in range(2 * npairs * CH, EPW, CH):
      cg0, cf0 = fire(off, bg0, bf0, sem0)
      for c in drain(off, bg0, bf0, cg0, cf0):
        c.wait()

  return gather_k


@functools.cache
def _make_scatter(e, with_msg):
  outs = []
  if with_msg:
    outs.append(jax.ShapeDtypeStruct((NC, N, e), jnp.float32))
  outs.append(jax.ShapeDtypeStruct((NC, N, 16), jnp.float32))
  outs.append(jax.ShapeDtypeStruct((8,), jnp.float32))
  scratch = [pltpu.VMEM((8,), jnp.float32),
             pltpu.VMEM((CH,), jnp.int32), pltpu.VMEM((CH,), jnp.int32)]
  if with_msg:
    scratch += [pltpu.VMEM((CH, e), jnp.float32),
                pltpu.VMEM((CH, e), jnp.float32)]
  scratch += [pltpu.VMEM((CH, 16), jnp.float32),
              pltpu.VMEM((CH, 16), jnp.float32)]
  if with_msg:
    scratch.append(pltpu.VMEM_SHARED((N, e), jnp.float32))
  scratch += [pltpu.VMEM_SHARED((N, 16), jnp.float32),
              pltpu.SemaphoreType.DMA, pltpu.SemaphoreType.DMA]

  @functools.partial(pl.kernel, out_type=tuple(outs), mesh=_mesh(),
                     scratch_types=scratch, compiler_params=_SC_PARAMS)
  def scatter_k(*refs):
    if with_msg:
      (msgP_h, posP_h, dst_h, zm_h, zp_h, tok_h, accM_h, accP_h, tok_o,
       tokb, bi0, bi1, bm0, bm1, bp0, bp1, spM, spP, sem0, sem1) = refs
    else:
      (posP_h, dst_h, zp_h, tok_h, accP_h, tok_o,
       tokb, bi0, bi1, bp0, bp1, spP, sem0, sem1) = refs
      msgP_h = spM = bm0 = bm1 = None
    cid = lax.axis_index("c")
    sid = lax.axis_index("s")
    wid = sid * NC + cid
    rbase = sid * RPT

    @pl.when(wid == 0)
    def _():
      pltpu.sync_copy(tok_h, tokb)
      pltpu.sync_copy(tokb, tok_o)
    pltpu.sync_copy(zp_h, spP.at[pl.ds(rbase, RPT)])
    if with_msg:
      pltpu.sync_copy(zm_h, spM.at[pl.ds(rbase, RPT)])
    plsc.subcore_barrier()
    base0 = wid * EPW
    npairs = EPW // (2 * CH)

    def fire(off, bi, bm, bp, sem):
      sl = pl.ds(base0 + off, CH)
      cs = [pltpu.async_copy(dst_h.at[sl], bi, sem),
            pltpu.async_copy(posP_h.at[sl], bp, sem)]
      if with_msg:
        cs.append(pltpu.async_copy(msgP_h.at[sl], bm, sem))
      return cs

    def scat(bi, bm, bp, cs):
      for c in cs:
        c.wait()
      pltpu.sync_copy(bp, spP.at[bi], add=True)
      if with_msg:
        pltpu.sync_copy(bm, spM.at[bi], add=True)

    def body(pi, carry):
      off0 = 2 * pi * CH
      c0 = fire(off0, bi0, bm0, bp0, sem0)
      c1 = fire(off0 + CH, bi1, bm1, bp1, sem1)
      scat(bi0, bm0, bp0, c0)
      scat(bi1, bm1, bp1, c1)
      return carry

    lax.fori_loop(0, npairs, body, 0)
    for off in range(2 * npairs * CH, EPW, CH):
      c0 = fire(off, bi0, bm0, bp0, sem0)
      scat(bi0, bm0, bp0, c0)
    plsc.subcore_barrier()
    pltpu.sync_copy(spP.at[pl.ds(rbase, RPT)],
                    accP_h.at[cid, pl.ds(rbase, RPT)])
    if with_msg:
      pltpu.sync_copy(spM.at[pl.ds(rbase, RPT)],
                      accM_h.at[cid, pl.ds(rbase, RPT)])

  return scatter_k


def _gather_edges(hx, src, dst, w, tok):
  hI, hJ, pI, pJ, tok2 = _make_gather(w)(hx, src, dst, tok)
  return (hI, hJ, pI, pJ), tok2


def _scatter_edges(msgP, posP, dst, e, tok):
  zp = jnp.zeros((RPT, 16), jnp.float32)
  if msgP is not None:
    zm = jnp.zeros((RPT, e), jnp.float32)
    accM, accP, tok2 = _make_scatter(e, True)(msgP, posP, dst, zm, zp, tok)
    return accM, accP, tok2
  accP, tok2 = _make_scatter(e, False)(posP, dst, zp, tok)
  return None, accP, tok2


def _bn_affine(st, g, b, denom):
  m = st[0:1, :]
  v = st[1:2, :] / denom
  scale = g * lax.rsqrt(v + EPS)
  return scale, b - m * scale


def _acc_stats(st_ref, z, nb):
  m_b = jnp.sum(z, 0, keepdims=True) / float(nb)
  zc = z - m_b
  ssw = jnp.sum(zc * zc, 0, keepdims=True)
  i = pl.program_id(0)

  @pl.when(i == 0)
  def _():
    st_ref[...] = jnp.concatenate([m_b, ssw], 0)

  @pl.when(i > 0)
  def _():
    n_prev = (i * nb).astype(jnp.float32)
    n_new = n_prev + float(nb)
    mean_prev = st_ref[0:1, :]
    delta = m_b - mean_prev
    st_ref[0:1, :] = mean_prev + delta * (float(nb) / n_new)
    st_ref[1:2, :] += ssw + delta * delta * (n_prev * float(nb) / n_new)


def _z1_block(W1_ref, b1_ref, hI_ref, hJ_ref, pI_ref, pJ_ref):
  d = pI_ref[...] - pJ_ref[...]
  dist = jnp.sqrt(jnp.sum(d[:, 0:3] * d[:, 0:3], axis=1, keepdims=True))
  msg = jnp.concatenate([hI_ref[...], hJ_ref[...], dist], axis=1)
  return (jnp.dot(msg, W1_ref[...], preferred_element_type=jnp.float32)
          + b1_ref[...]), d


def _eblk(e):
  return pl.BlockSpec((BE, e), lambda i: (i, 0))


def _cblk(r, e):
  return pl.BlockSpec((r, e), lambda i: (0, 0))


@functools.cache
def _make_k1s(e):
  def body(W1, b1, hI, hJ, pI, pJ, z1_out, pd_out, st):
    z1, d = _z1_block(W1, b1, hI, hJ, pI, pJ)
    z1_out[...] = z1
    pd_out[...] = d
    _acc_stats(st, z1, BE)

  return pl.pallas_call(
      body, grid=(GRID,),
      in_specs=[_cblk(2 * e + 1, e), _cblk(1, e),
                _eblk(e), _eblk(e), _eblk(8), _eblk(8)],
      out_specs=(_eblk(e), _eblk(8), _cblk(2, e)),
      out_shape=(jax.ShapeDtypeStruct((E, e), jnp.float32),
                 jax.ShapeDtypeStruct((E, 8), jnp.float32),
                 jax.ShapeDtypeStruct((2, e), jnp.float32)))


@functools.cache
def _make_k2(e):
  def body(st1, g1, bb1, W2, b2, z1, z2, st2):
    scale, shift = _bn_affine(st1[...], g1[...], bb1[...], float(E))
    a1 = jnp.maximum(z1[...] * scale + shift, 0.0)
    z2v = jnp.dot(a1, W2[...], preferred_element_type=jnp.float32) + b2[...]
    z2[...] = z2v
    _acc_stats(st2, z2v, BE)

  return pl.pallas_call(
      body, grid=(GRID,),
      in_specs=[_cblk(2, e), _cblk(1, e), _cblk(1, e),
                _cblk(e, e), _cblk(1, e), _eblk(e)],
      out_specs=(_eblk(e), _cblk(2, e)),
      out_shape=(jax.ShapeDtypeStruct((E, e), jnp.float32),
                 jax.ShapeDtypeStruct((2, e), jnp.float32)))


@functools.cache
def _make_k3(e):
  def body(st2, g2, b2, Wp1, bp1, z2, z3, st3):
    scale, shift = _bn_affine(st2[...], g2[...], b2[...], float(E))
    msg = jnp.maximum(z2[...] * scale + shift, 0.0)
    z3v = jnp.dot(msg, Wp1[...], preferred_element_type=jnp.float32) + bp1[...]
    z3[...] = z3v
    _acc_stats(st3, z3v, BE)

  return pl.pallas_call(
      body, grid=(GRID,),
      in_specs=[_cblk(2, e), _cblk(1, e), _cblk(1, e),
                _cblk(e, e), _cblk(1, e), _eblk(e)],
      out_specs=(_eblk(e), _cblk(2, e)),
      out_shape=(jax.ShapeDtypeStruct((E, e), jnp.float32),
                 jax.ShapeDtypeStruct((2, e), jnp.float32)))


@functools.cache
def _make_k4(e, with_msg):
  def body(*refs):
    if with_msg:
      (st2, g2, b2, st3, g3, b3, wp2, bp2, z2, z3, pd, *pay) = refs
    else:
      (st3, g3, b3, wp2, bp2, z3, pd, pay) = refs
    scale3, shift3 = _bn_affine(st3[...], g3[...], b3[...], float(E))
    a3 = jnp.maximum(z3[...] * scale3 + shift3, 0.0)
    w = jnp.dot(a3, wp2[...], preferred_element_type=jnp.float32) + bp2[...]
    wpos = pd[...][:, 0:3] * w
    tail = jnp.concatenate(
        [wpos, jnp.ones((BE, 1), jnp.float32),
         jnp.zeros((BE, 12), jnp.float32)], axis=1)
    if with_msg:
      scale2, shift2 = _bn_affine(st2[...], g2[...], b2[...], float(E))
      msg = jnp.maximum(z2[...] * scale2 + shift2, 0.0)
      msgP, posP = pay
      msgP[...] = msg
      posP[...] = tail
    else:
      pay[...] = tail

  if with_msg:
    in_specs = [_cblk(2, e), _cblk(1, e), _cblk(1, e),
                _cblk(2, e), _cblk(1, e), _cblk(1, e),
                _cblk(e, 1), _cblk(1, 1),
                _eblk(e), _eblk(e), _eblk(8)]
    out_specs = (_eblk(e), _eblk(16))
    out_shape = (jax.ShapeDtypeStruct((E, e), jnp.float32),
                 jax.ShapeDtypeStruct((E, 16), jnp.float32))
  else:
    in_specs = [_cblk(2, e), _cblk(1, e), _cblk(1, e),
                _cblk(e, 1), _cblk(1, 1),
                _eblk(e), _eblk(8)]
    out_specs = _eblk(16)
    out_shape = jax.ShapeDtypeStruct((E, 16), jnp.float32)

  return pl.pallas_call(body, grid=(GRID,), in_specs=in_specs,
                        out_specs=out_specs, out_shape=out_shape)


BNODE = 2000
NGRID = N // BNODE


def _nblk(e):
  return pl.BlockSpec((BNODE, e), lambda i: (i, 0))


def _nblk3(e):
  return pl.BlockSpec((NC, BNODE, e), lambda i: (0, i, 0))


def _ncblk(r, e):
  return pl.BlockSpec((r, e), lambda i: (0, 0))


def _dot(a, b):
  return jnp.dot(a, b, preferred_element_type=jnp.float32)


@functools.cache
def _make_node_a(e):
  def body(h, pos4, accM, accP, Wu1, bu1, z, pos4_out, st):
    aPc = accP[...]
    aP = aPc[0] + aPc[1]
    cnt = jnp.maximum(aP[:, 3:4], 1.0)
    aMc = accM[...]
    aM = (aMc[0] + aMc[1]) / cnt
    zv = _dot(jnp.concatenate([h[...], aM], axis=1), Wu1[...]) + bu1[...]
    z[...] = zv
    pos4_out[...] = pos4[...] + jnp.concatenate(
        [aP[:, 0:3] / cnt, jnp.zeros((BNODE, 5), jnp.float32)], axis=1)
    _acc_stats(st, zv, BNODE)

  return pl.pallas_call(
      body, grid=(NGRID,),
      in_specs=[_nblk(e), _nblk(8), _nblk3(e), _nblk3(16),
                _ncblk(2 * e, e), _ncblk(1, e)],
      out_specs=(_nblk(e), _nblk(8), _ncblk(2, e)),
      out_shape=(jax.ShapeDtypeStruct((N, e), jnp.float32),
                 jax.ShapeDtypeStruct((N, 8), jnp.float32),
                 jax.ShapeDtypeStruct((2, e), jnp.float32)))


@functools.cache
def _make_node_b(e):
  def body(st1, g1, b1, Wu2, bu2, z, z2, st2):
    scale, shift = _bn_affine(st1[...], g1[...], b1[...], float(N))
    u = jnp.maximum(z[...] * scale + shift, 0.0)
    z2v = _dot(u, Wu2[...]) + bu2[...]
    z2[...] = z2v
    _acc_stats(st2, z2v, BNODE)

  return pl.pallas_call(
      body, grid=(NGRID,),
      in_specs=[_ncblk(2, e), _ncblk(1, e), _ncblk(1, e),
                _ncblk(e, e), _ncblk(1, e), _nblk(e)],
      out_specs=(_nblk(e), _ncblk(2, e)),
      out_shape=(jax.ShapeDtypeStruct((N, e), jnp.float32),
                 jax.ShapeDtypeStruct((2, e), jnp.float32)))


@functools.cache
def _make_node_c(e, e_out):
  def body(st2, g2, b2, Wo, bo, z2, h_out):
    scale, shift = _bn_affine(st2[...], g2[...], b2[...], float(N))
    u2 = jnp.maximum(z2[...] * scale + shift, 0.0)
    h_out[...] = _dot(u2, Wo[...]) + bo[...]

  return pl.pallas_call(
      body, grid=(NGRID,),
      in_specs=[_ncblk(2, e), _ncblk(1, e), _ncblk(1, e),
                _ncblk(e, e_out), _ncblk(1, e_out), _nblk(e)],
      out_specs=_nblk(e_out),
      out_shape=jax.ShapeDtypeStruct((N, e_out), jnp.float32))


@functools.cache
def _make_posnode():
  def body(pos4, accP, pos4_out):
    aPc = accP[...]
    aP = aPc[0] + aPc[1]
    cnt = jnp.maximum(aP[:, 3:4], 1.0)
    pos4_out[...] = pos4[...] + jnp.concatenate(
        [aP[:, 0:3] / cnt, jnp.zeros((N, 5), jnp.float32)], axis=1)

  return pl.pallas_call(
      body, out_shape=jax.ShapeDtypeStruct((N, 8), jnp.float32))


@functools.cache
def _make_init(e0):
  def body(x, Wi, bi, h0):
    h0[...] = x[...] * Wi[...] + bi[...]

  return pl.pallas_call(
      body, grid=(NGRID,),
      in_specs=[_nblk(1), _ncblk(1, 64), _ncblk(1, 64)],
      out_specs=_nblk(64),
      out_shape=jax.ShapeDtypeStruct((N, 64), jnp.float32))


_DIMS = [64, 128, 256]


def _row(a):
  return a.reshape(1, -1)


def _run_pair(gs):
  for g in gs:
    g["pos4"] = jnp.pad(g["pos"], ((0, 0), (0, 5)))
    g["src"] = g["ei"][0]
    g["dst"] = g["ei"][1]
    g["h"] = _make_init(64)(g["x"], g["lin"]["W"], _row(g["lin"]["b"]))
  gs[0]["tok"] = jnp.zeros((8,), jnp.float32)
  nl = len(gs[0]["layers"])
  for li in range(nl):
    e = _DIMS[li]
    last = li == nl - 1
    for g in gs:
      hx = jnp.concatenate([g["h"], g["pos4"]], axis=1)
      g["gath"], gs[0]["tok"] = _gather_edges(
          hx, g["src"], g["dst"], e + 8, gs[0]["tok"])
    for g in gs:
      lp = g["layers"][li]
      hI, hJ, pI, pJ = g["gath"]
      g["z1"], g["pd"], g["st1"] = _make_k1s(e)(
          lp["msg1"]["W"], _row(lp["msg1"]["b"]), hI, hJ, pI, pJ)
    for g in gs:
      lp = g["layers"][li]
      g["z2"], g["st2"] = _make_k2(e)(
          g["st1"], _row(lp["msg_bn1"]["g"]), _row(lp["msg_bn1"]["b"]),
          lp["msg2"]["W"], _row(lp["msg2"]["b"]), g["z1"])
    for g in gs:
      lp = g["layers"][li]
      g["z3"], g["st3"] = _make_k3(e)(
          g["st2"], _row(lp["msg_bn2"]["g"]), _row(lp["msg_bn2"]["b"]),
          lp["pos1"]["W"], _row(lp["pos1"]["b"]), g["z2"])
    for g in gs:
      lp = g["layers"][li]
      bn2g, bn2b = _row(lp["msg_bn2"]["g"]), _row(lp["msg_bn2"]["b"])
      bnpg, bnpb = _row(lp["pos_bn"]["g"]), _row(lp["pos_bn"]["b"])
      wp2 = lp["pos2"]["W"]
      bp2 = lp["pos2"]["b"].reshape(1, 1)
      if not last:
        g["msgP"], g["posP"] = _make_k4(e, True)(
            g["st2"], bn2g, bn2b, g["st3"], bnpg, bnpb,
            wp2, bp2, g["z2"], g["z3"], g["pd"])
      else:
        g["msgP"] = None
        g["posP"] = _make_k4(e, False)(
            g["st3"], bnpg, bnpb, wp2, bp2, g["z3"], g["pd"])
    for g in gs:
      g["accM"], g["accP"], gs[0]["tok"] = _scatter_edges(
          g["msgP"], g["posP"], g["dst"], e, gs[0]["tok"])
    if not last:
      e_out = _DIMS[li + 1]
      for g in gs:
        lp = g["layers"][li]
        g["z"], g["pos4"], g["stu1"] = _make_node_a(e)(
            g["h"], g["pos4"], g["accM"], g["accP"],
            lp["upd1"]["W"], _row(lp["upd1"]["b"]))
      for g in gs:
        lp = g["layers"][li]
        g["z2n"], g["stu2"] = _make_node_b(e)(
            g["stu1"], _row(lp["upd_bn1"]["g"]), _row(lp["upd_bn1"]["b"]),
            lp["upd2"]["W"], _row(lp["upd2"]["b"]), g["z"])
      for g in gs:
        lp = g["layers"][li]
        g["h"] = _make_node_c(e, e_out)(
            g["stu2"], _row(lp["upd_bn2"]["g"]), _row(lp["upd_bn2"]["b"]),
            lp["out"]["W"], _row(lp["out"]["b"]), g["z2n"])
    else:
      for g in gs:
        g["pos4"] = _make_posnode()(g["pos4"], g["accP"])
  return tuple(g["pos4"][:, :3] for g in gs)


def kernel(rec_x, rec_pos, rec_edge_index, lig_x, lig_pos, lig_edge_index,
           params):
  gs = [
      dict(x=rec_x, pos=rec_pos, ei=rec_edge_index,
           lin=params["lin_in_rec"], layers=params["rec_layers"]),
      dict(x=lig_x, pos=lig_pos, ei=lig_edge_index,
           lin=params["lin_in_lig"], layers=params["lig_layers"]),
  ]
  p_r, p_l = _run_pair(gs)
  return (p_r, p_l)


# speedup vs baseline: 1.0431x; 1.0431x over previous
import functools

import jax
import jax.numpy as jnp
from jax import lax
from jax.experimental import pallas as pl
from jax.experimental.pallas import tpu as pltpu
from jax.experimental.pallas import tpu_sc as plsc

N = 10000
E = 320000
NC = 2
NS = 16
NW = NC * NS
EPW = E // NW
CH = 80
RPT = N // NS
BE = 4000
GRID = E // BE
EPS = 1e-5

_mesh = functools.partial(
    plsc.VectorSubcoreMesh,
    core_axis_name="c", subcore_axis_name="s", num_cores=NC, num_subcores=NS)

_SC_PARAMS = pltpu.CompilerParams(use_tc_tiling_on_sc=False)


@functools.cache
def _make_gather(w):
  e = w - 8

  @functools.partial(
      pl.kernel,
      out_type=(jax.ShapeDtypeStruct((E, e), jnp.float32),
                jax.ShapeDtypeStruct((E, e), jnp.float32),
                jax.ShapeDtypeStruct((E, 8), jnp.float32),
                jax.ShapeDtypeStruct((E, 8), jnp.float32),
                jax.ShapeDtypeStruct((8,), jnp.float32)),
      mesh=_mesh(),
      scratch_types=[pltpu.VMEM((EPW,), jnp.int32),
                     pltpu.VMEM((EPW,), jnp.int32),
                     pltpu.VMEM((CH, w), jnp.float32),
                     pltpu.VMEM((CH, w), jnp.float32),
                     pltpu.VMEM((CH, w), jnp.float32),
                     pltpu.VMEM((CH, w), jnp.float32),
                     pltpu.VMEM((8,), jnp.float32),
                     pltpu.SemaphoreType.DMA,
                     pltpu.SemaphoreType.DMA,
                     pltpu.SemaphoreType.DMA],
      compiler_params=_SC_PARAMS)
  def gather_k(t_h, src_h, dst_h, tok_h, hI_h, hJ_h, pI_h, pJ_h, tok_o,
               idxd, idxs, bg0, bf0, bg1, bf1, tokb, sem0, sem1, semw):
    wid = lax.axis_index("s") * NC + lax.axis_index("c")
    base0 = wid * EPW

    @pl.when(wid == 0)
    def _():
      pltpu.sync_copy(tok_h, tokb)
      pltpu.sync_copy(tokb, tok_o)
    pltpu.sync_copy(dst_h.at[pl.ds(base0, EPW)], idxd)
    pltpu.sync_copy(src_h.at[pl.ds(base0, EPW)], idxs)
    npairs = EPW // (2 * CH)

    def fire(off, bg, bf, sem):
      cg = pltpu.async_copy(t_h.at[idxd.at[pl.ds(off, CH)]], bg, sem)
      cf = pltpu.async_copy(t_h.at[idxs.at[pl.ds(off, CH)]], bf, sem)
      return cg, cf

    def drain(off, bg, bf, cg, cf):
      cg.wait()
      cf.wait()
      sl = pl.ds(base0 + off, CH)
      wg = pltpu.async_copy(bg.at[:, pl.ds(0, e)], hI_h.at[sl], semw)
      wf = pltpu.async_copy(bf.at[:, pl.ds(0, e)], hJ_h.at[sl], semw)
      wp = pltpu.async_copy(bg.at[:, pl.ds(e, 8)], pI_h.at[sl], semw)
      wq = pltpu.async_copy(bf.at[:, pl.ds(e, 8)], pJ_h.at[sl], semw)
      return wg, wf, wp, wq

    def body(pi, carry):
      off0 = 2 * pi * CH
      off1 = off0 + CH
      cg0, cf0 = fire(off0, bg0, bf0, sem0)
      cg1, cf1 = fire(off1, bg1, bf1, sem1)
      ws0 = drain(off0, bg0, bf0, cg0, cf0)
      ws1 = drain(off1, bg1, bf1, cg1, cf1)
      for c in ws0 + ws1:
        c.wait()
      return carry

    lax.fori_loop(0, npairs, body, 0)
    for off in range(2 * npairs * CH, EPW, CH):
      cg0, cf0 = fire(off, bg0, bf0, sem0)
      for c in drain(off, bg0, bf0, cg0, cf0):
        c.wait()

  return gather_k


@functools.cache
def _make_scatter(e, with_msg):
  outs = []
  if with_msg:
    outs.append(jax.ShapeDtypeStruct((NC, N, e), jnp.float32))
  outs.append(jax.ShapeDtypeStruct((NC, N, 16), jnp.float32))
  outs.append(jax.ShapeDtypeStruct((8,), jnp.float32))
  scratch = [pltpu.VMEM((8,), jnp.float32),
             pltpu.VMEM((CH,), jnp.int32), pltpu.VMEM((CH,), jnp.int32)]
  if with_msg:
    scratch += [pltpu.VMEM((CH, e), jnp.float32),
                pltpu.VMEM((CH, e), jnp.float32)]
  scratch += [pltpu.VMEM((CH, 16), jnp.float32),
              pltpu.VMEM((CH, 16), jnp.float32)]
  if with_msg:
    scratch.append(pltpu.VMEM_SHARED((N, e), jnp.float32))
  scratch += [pltpu.VMEM_SHARED((N, 16), jnp.float32),
              pltpu.SemaphoreType.DMA, pltpu.SemaphoreType.DMA]

  @functools.partial(pl.kernel, out_type=tuple(outs), mesh=_mesh(),
                     scratch_types=scratch, compiler_params=_SC_PARAMS)
  def scatter_k(*refs):
    if with_msg:
      (msgP_h, posP_h, dst_h, zm_h, zp_h, tok_h, accM_h, accP_h, tok_o,
       tokb, bi0, bi1, bm0, bm1, bp0, bp1, spM, spP, sem0, sem1) = refs
    else:
      (posP_h, dst_h, zp_h, tok_h, accP_h, tok_o,
       tokb, bi0, bi1, bp0, bp1, spP, sem0, sem1) = refs
      msgP_h = spM = bm0 = bm1 = None
    cid = lax.axis_index("c")
    sid = lax.axis_index("s")
    wid = sid * NC + cid
    rbase = sid * RPT

    @pl.when(wid == 0)
    def _():
      pltpu.sync_copy(tok_h, tokb)
      pltpu.sync_copy(tokb, tok_o)
    pltpu.sync_copy(zp_h, spP.at[pl.ds(rbase, RPT)])
    if with_msg:
      pltpu.sync_copy(zm_h, spM.at[pl.ds(rbase, RPT)])
    plsc.subcore_barrier()
    base0 = wid * EPW
    npairs = EPW // (2 * CH)

    def fire(off, bi, bm, bp, sem):
      sl = pl.ds(base0 + off, CH)
      cs = [pltpu.async_copy(dst_h.at[sl], bi, sem),
            pltpu.async_copy(posP_h.at[sl], bp, sem)]
      if with_msg:
        cs.append(pltpu.async_copy(msgP_h.at[sl], bm, sem))
      return cs

    def scat(bi, bm, bp, cs):
      for c in cs:
        c.wait()
      pltpu.sync_copy(bp, spP.at[bi], add=True)
      if with_msg:
        pltpu.sync_copy(bm, spM.at[bi], add=True)

    def body(pi, carry):
      off0 = 2 * pi * CH
      c0 = fire(off0, bi0, bm0, bp0, sem0)
      c1 = fire(off0 + CH, bi1, bm1, bp1, sem1)
      scat(bi0, bm0, bp0, c0)
      scat(bi1, bm1, bp1, c1)
      return carry

    lax.fori_loop(0, npairs, body, 0)
    for off in range(2 * npairs * CH, EPW, CH):
      c0 = fire(off, bi0, bm0, bp0, sem0)
      scat(bi0, bm0, bp0, c0)
    plsc.subcore_barrier()
    pltpu.sync_copy(spP.at[pl.ds(rbase, RPT)],
                    accP_h.at[cid, pl.ds(rbase, RPT)])
    if with_msg:
      pltpu.sync_copy(spM.at[pl.ds(rbase, RPT)],
                      accM_h.at[cid, pl.ds(rbase, RPT)])

  return scatter_k


def _gather_edges(hx, src, dst, w, tok):
  hI, hJ, pI, pJ, tok2 = _make_gather(w)(hx, src, dst, tok)
  return (hI, hJ, pI, pJ), tok2


def _scatter_edges(msgP, posP, dst, e, tok):
  zp = jnp.zeros((RPT, 16), jnp.float32)
  if msgP is not None:
    zm = jnp.zeros((RPT, e), jnp.float32)
    accM, accP, tok2 = _make_scatter(e, True)(msgP, posP, dst, zm, zp, tok)
    return accM, accP, tok2
  accP, tok2 = _make_scatter(e, False)(posP, dst, zp, tok)
  return None, accP, tok2


def _bn_affine(st, g, b, denom):
  m = st[0:1, :]
  v = st[1:2, :] / denom
  scale = g * lax.rsqrt(v + EPS)
  return scale, b - m * scale


def _acc_stats(st_ref, z, nb):
  m_b = jnp.sum(z, 0, keepdims=True) / float(nb)
  zc = z - m_b
  ssw = jnp.sum(zc * zc, 0, keepdims=True)
  i = pl.program_id(0)

  @pl.when(i == 0)
  def _():
    st_ref[...] = jnp.concatenate([m_b, ssw], 0)

  @pl.when(i > 0)
  def _():
    n_prev = (i * nb).astype(jnp.float32)
    n_new = n_prev + float(nb)
    mean_prev = st_ref[0:1, :]
    delta = m_b - mean_prev
    st_ref[0:1, :] = mean_prev + delta * (float(nb) / n_new)
    st_ref[1:2, :] += ssw + delta * delta * (n_prev * float(nb) / n_new)


def _z1_block(W1_ref, b1_ref, hI_ref, hJ_ref, pI_ref, pJ_ref):
  d = pI_ref[...] - pJ_ref[...]
  dist = jnp.sqrt(jnp.sum(d[:, 0:3] * d[:, 0:3], axis=1, keepdims=True))
  msg = jnp.concatenate([hI_ref[...], hJ_ref[...], dist], axis=1)
  return (jnp.dot(msg, W1_ref[...], preferred_element_type=jnp.float32)
          + b1_ref[...]), d


def _eblk(e):
  return pl.BlockSpec((BE, e), lambda i: (i, 0))


def _cblk(r, e):
  return pl.BlockSpec((r, e), lambda i: (0, 0))


@functools.cache
def _make_k1s(e):
  def body(W1, b1, hI, hJ, pI, pJ, z1_out, pd_out, st):
    z1, d = _z1_block(W1, b1, hI, hJ, pI, pJ)
    z1_out[...] = z1
    pd_out[...] = d
    _acc_stats(st, z1, BE)

  return pl.pallas_call(
      body, grid=(GRID,),
      in_specs=[_cblk(2 * e + 1, e), _cblk(1, e),
                _eblk(e), _eblk(e), _eblk(8), _eblk(8)],
      out_specs=(_eblk(e), _eblk(8), _cblk(2, e)),
      out_shape=(jax.ShapeDtypeStruct((E, e), jnp.float32),
                 jax.ShapeDtypeStruct((E, 8), jnp.float32),
                 jax.ShapeDtypeStruct((2, e), jnp.float32)))


@functools.cache
def _make_k2(e):
  def body(st1, g1, bb1, W2, b2, z1, z2, st2):
    scale, shift = _bn_affine(st1[...], g1[...], bb1[...], float(E))
    a1 = jnp.maximum(z1[...] * scale + shift, 0.0)
    z2v = jnp.dot(a1, W2[...], preferred_element_type=jnp.float32) + b2[...]
    z2[...] = z2v
    _acc_stats(st2, z2v, BE)

  return pl.pallas_call(
      body, grid=(GRID,),
      in_specs=[_cblk(2, e), _cblk(1, e), _cblk(1, e),
                _cblk(e, e), _cblk(1, e), _eblk(e)],
      out_specs=(_eblk(e), _cblk(2, e)),
      out_shape=(jax.ShapeDtypeStruct((E, e), jnp.float32),
                 jax.ShapeDtypeStruct((2, e), jnp.float32)))


@functools.cache
def _make_k3(e):
  def body(st2, g2, b2, Wp1, bp1, z2, z3, st3):
    scale, shift = _bn_affine(st2[...], g2[...], b2[...], float(E))
    msg = jnp.maximum(z2[...] * scale + shift, 0.0)
    z3v = jnp.dot(msg, Wp1[...], preferred_element_type=jnp.float32) + bp1[...]
    z3[...] = z3v
    _acc_stats(st3, z3v, BE)

  return pl.pallas_call(
      body, grid=(GRID,),
      in_specs=[_cblk(2, e), _cblk(1, e), _cblk(1, e),
                _cblk(e, e), _cblk(1, e), _eblk(e)],
      out_specs=(_eblk(e), _cblk(2, e)),
      out_shape=(jax.ShapeDtypeStruct((E, e), jnp.float32),
                 jax.ShapeDtypeStruct((2, e), jnp.float32)))


@functools.cache
def _make_k4(e, with_msg):
  def body(*refs):
    if with_msg:
      (st2, g2, b2, st3, g3, b3, wp2, bp2, z2, z3, pd, *pay) = refs
    else:
      (st3, g3, b3, wp2, bp2, z3, pd, pay) = refs
    scale3, shift3 = _bn_affine(st3[...], g3[...], b3[...], float(E))
    a3 = jnp.maximum(z3[...] * scale3 + shift3, 0.0)
    w = jnp.dot(a3, wp2[...], preferred_element_type=jnp.float32) + bp2[...]
    wpos = pd[...][:, 0:3] * w
    tail = jnp.concatenate(
        [wpos, jnp.ones((BE, 1), jnp.float32),
         jnp.zeros((BE, 12), jnp.float32)], axis=1)
    if with_msg:
      scale2, shift2 = _bn_affine(st2[...], g2[...], b2[...], float(E))
      msg = jnp.maximum(z2[...] * scale2 + shift2, 0.0)
      msgP, posP = pay
      msgP[...] = msg
      posP[...] = tail
    else:
      pay[...] = tail

  if with_msg:
    in_specs = [_cblk(2, e), _cblk(1, e), _cblk(1, e),
                _cblk(2, e), _cblk(1, e), _cblk(1, e),
                _cblk(e, 1), _cblk(1, 1),
                _eblk(e), _eblk(e), _eblk(8)]
    out_specs = (_eblk(e), _eblk(16))
    out_shape = (jax.ShapeDtypeStruct((E, e), jnp.float32),
                 jax.ShapeDtypeStruct((E, 16), jnp.float32))
  else:
    in_specs = [_cblk(2, e), _cblk(1, e), _cblk(1, e),
                _cblk(e, 1), _cblk(1, 1),
                _eblk(e), _eblk(8)]
    out_specs = _eblk(16)
    out_shape = jax.ShapeDtypeStruct((E, 16), jnp.float32)

  return pl.pallas_call(body, grid=(GRID,), in_specs=in_specs,
                        out_specs=out_specs, out_shape=out_shape)


BNODE = 2000
NGRID = N // BNODE


def _nblk(e):
  return pl.BlockSpec((BNODE, e), lambda i: (i, 0))


def _nblk3(e):
  return pl.BlockSpec((NC, BNODE, e), lambda i: (0, i, 0))


def _ncblk(r, e):
  return pl.BlockSpec((r, e), lambda i: (0, 0))


def _dot(a, b):
  return jnp.dot(a, b, preferred_element_type=jnp.float32)


@functools.cache
def _make_node_a(e):
  def body(h, pos4, accM, accP, Wu1, bu1, z, pos4_out, st):
    aPc = accP[...]
    aP = aPc[0] + aPc[1]
    cnt = jnp.maximum(aP[:, 3:4], 1.0)
    aMc = accM[...]
    aM = (aMc[0] + aMc[1]) / cnt
    zv = _dot(jnp.concatenate([h[...], aM], axis=1), Wu1[...]) + bu1[...]
    z[...] = zv
    pos4_out[...] = pos4[...] + jnp.concatenate(
        [aP[:, 0:3] / cnt, jnp.zeros((BNODE, 5), jnp.float32)], axis=1)
    _acc_stats(st, zv, BNODE)

  return pl.pallas_call(
      body, grid=(NGRID,),
      in_specs=[_nblk(e), _nblk(8), _nblk3(e), _nblk3(16),
                _ncblk(2 * e, e), _ncblk(1, e)],
      out_specs=(_nblk(e), _nblk(8), _ncblk(2, e)),
      out_shape=(jax.ShapeDtypeStruct((N, e), jnp.float32),
                 jax.ShapeDtypeStruct((N, 8), jnp.float32),
                 jax.ShapeDtypeStruct((2, e), jnp.float32)))


@functools.cache
def _make_node_b(e):
  def body(st1, g1, b1, Wu2, bu2, z, z2, st2):
    scale, shift = _bn_affine(st1[...], g1[...], b1[...], float(N))
    u = jnp.maximum(z[...] * scale + shift, 0.0)
    z2v = _dot(u, Wu2[...]) + bu2[...]
    z2[...] = z2v
    _acc_stats(st2, z2v, BNODE)

  return pl.pallas_call(
      body, grid=(NGRID,),
      in_specs=[_ncblk(2, e), _ncblk(1, e), _ncblk(1, e),
                _ncblk(e, e), _ncblk(1, e), _nblk(e)],
      out_specs=(_nblk(e), _ncblk(2, e)),
      out_shape=(jax.ShapeDtypeStruct((N, e), jnp.float32),
                 jax.ShapeDtypeStruct((2, e), jnp.float32)))


@functools.cache
def _make_node_c(e, e_out):
  def body(st2, g2, b2, Wo, bo, z2, h_out):
    scale, shift = _bn_affine(st2[...], g2[...], b2[...], float(N))
    u2 = jnp.maximum(z2[...] * scale + shift, 0.0)
    h_out[...] = _dot(u2, Wo[...]) + bo[...]

  return pl.pallas_call(
      body, grid=(NGRID,),
      in_specs=[_ncblk(2, e), _ncblk(1, e), _ncblk(1, e),
                _ncblk(e, e_out), _ncblk(1, e_out), _nblk(e)],
      out_specs=_nblk(e_out),
      out_shape=jax.ShapeDtypeStruct((N, e_out), jnp.float32))


@functools.cache
def _make_posnode():
  def body(pos4, accP, pos4_out):
    aPc = accP[...]
    aP = aPc[0] + aPc[1]
    cnt = jnp.maximum(aP[:, 3:4], 1.0)
    pos4_out[...] = pos4[...] + jnp.concatenate(
        [aP[:, 0:3] / cnt, jnp.zeros((N, 5), jnp.float32)], axis=1)

  return pl.pallas_call(
      body, out_shape=jax.ShapeDtypeStruct((N, 8), jnp.float32))


@functools.cache
def _make_init(e0):
  def body(x, Wi, bi, h0):
    h0[...] = x[...] * Wi[...] + bi[...]

  return pl.pallas_call(
      body, grid=(NGRID,),
      in_specs=[_nblk(1), _ncblk(1, 64), _ncblk(1, 64)],
      out_specs=_nblk(64),
      out_shape=jax.ShapeDtypeStruct((N, 64), jnp.float32))


_DIMS = [64, 128, 256]


def _row(a):
  return a.reshape(1, -1)


def _run_pair(gs):
  for g in gs:
    g["pos4"] = jnp.pad(g["pos"], ((0, 0), (0, 5)))
    g["src"] = g["ei"][0]
    g["dst"] = g["ei"][1]
    g["h"] = _make_init(64)(g["x"], g["lin"]["W"], _row(g["lin"]["b"]))
  gs[0]["tok"] = jnp.zeros((8,), jnp.float32)
  nl = len(gs[0]["layers"])
  for li in range(nl):
    e = _DIMS[li]
    last = li == nl - 1
    for g in gs:
      hx = jnp.concatenate([g["h"], g["pos4"]], axis=1)
      g["gath"], gs[0]["tok"] = _gather_edges(
          hx, g["src"], g["dst"], e + 8, gs[0]["tok"])
    for g in gs:
      lp = g["layers"][li]
      hI, hJ, pI, pJ = g["gath"]
      g["z1"], g["pd"], g["st1"] = _make_k1s(e)(
          lp["msg1"]["W"], _row(lp["msg1"]["b"]), hI, hJ, pI, pJ)
    for g in gs:
      lp = g["layers"][li]
      g["z2"], g["st2"] = _make_k2(e)(
          g["st1"], _row(lp["msg_bn1"]["g"]), _row(lp["msg_bn1"]["b"]),
          lp["msg2"]["W"], _row(lp["msg2"]["b"]), g["z1"])
    for g in gs:
      lp = g["layers"][li]
      g["z3"], g["st3"] = _make_k3(e)(
          g["st2"], _row(lp["msg_bn2"]["g"]), _row(lp["msg_bn2"]["b"]),
          lp["pos1"]["W"], _row(lp["pos1"]["b"]), g["z2"])
    for g in gs:
      lp = g["layers"][li]
      bn2g, bn2b = _row(lp["msg_bn2"]["g"]), _row(lp["msg_bn2"]["b"])
      bnpg, bnpb = _row(lp["pos_bn"]["g"]), _row(lp["pos_bn"]["b"])
      wp2 = lp["pos2"]["W"]
      bp2 = lp["pos2"]["b"].reshape(1, 1)
      if not last:
        g["msgP"], g["posP"] = _make_k4(e, True)(
            g["st2"], bn2g, bn2b, g["st3"], bnpg, bnpb,
            wp2, bp2, g["z2"], g["z3"], g["pd"])
      else:
        g["msgP"] = None
        g["posP"] = _make_k4(e, False)(
            g["st3"], bnpg, bnpb, wp2, bp2, g["z3"], g["pd"])
    for g in gs:
      g["accM"], g["accP"], gs[0]["tok"] = _scatter_edges(
          g["msgP"], g["posP"], g["dst"], e, gs[0]["tok"])
    if not last:
      e_out = _DIMS[li + 1]
      for g in gs:
        lp = g["layers"][li]
        g["z"], g["pos4"], g["stu1"] = _make_node_a(e)(
            g["h"], g["pos4"], g["accM"], g["accP"],
            lp["upd1"]["W"], _row(lp["upd1"]["b"]))
      for g in gs:
        lp = g["layers"][li]
        g["z2n"], g["stu2"] = _make_node_b(e)(
            g["stu1"], _row(lp["upd_bn1"]["g"]), _row(lp["upd_bn1"]["b"]),
            lp["upd2"]["W"], _row(lp["upd2"]["b"]), g["z"])
      for g in gs:
        lp = g["layers"][li]
        g["h"] = _make_node_c(e, e_out)(
            g["stu2"], _row(lp["upd_bn2"]["g"]), _row(lp["upd_bn2"]["b"]),
            lp["out"]["W"], _row(lp["out"]["b"]), g["z2n"])
    else:
      for g in gs:
        g["pos4"] = _make_posnode()(g["pos4"], g["accP"])
  return tuple(g["pos4"][:, :3] for g in gs)


def kernel(rec_x, rec_pos, rec_edge_index, lig_x, lig_pos, lig_edge_index,
           params):
  gs = [
      dict(x=rec_x, pos=rec_pos, ei=rec_edge_index,
           lin=params["lin_in_rec"], layers=params["rec_layers"]),
      dict(x=lig_x, pos=lig_pos, ei=lig_edge_index,
           lin=params["lin_in_lig"], layers=params["lig_layers"]),
  ]
  p_r, p_l = _run_pair(gs)
  return (p_r, p_l)


# speedup vs baseline: 1.0536x; 1.0101x over previous
import functools

import jax
import jax.numpy as jnp
from jax import lax
from jax.experimental import pallas as pl
from jax.experimental.pallas import tpu as pltpu
from jax.experimental.pallas import tpu_sc as plsc

N = 10000
E = 320000
NC = 2
NS = 16
NW = NC * NS
EPW = E // NW
CH = 80
RPT = N // NS
BE = 5000
GRID = E // BE
EPS = 1e-5

_mesh = functools.partial(
    plsc.VectorSubcoreMesh,
    core_axis_name="c", subcore_axis_name="s", num_cores=NC, num_subcores=NS)

_SC_PARAMS = pltpu.CompilerParams(use_tc_tiling_on_sc=False)


@functools.cache
def _make_gather(w):
  e = w - 8

  @functools.partial(
      pl.kernel,
      out_type=(jax.ShapeDtypeStruct((E, e), jnp.float32),
                jax.ShapeDtypeStruct((E, e), jnp.float32),
                jax.ShapeDtypeStruct((E, 8), jnp.float32),
                jax.ShapeDtypeStruct((E, 8), jnp.float32),
                jax.ShapeDtypeStruct((8,), jnp.float32)),
      mesh=_mesh(),
      scratch_types=[pltpu.VMEM((EPW,), jnp.int32),
                     pltpu.VMEM((EPW,), jnp.int32),
                     pltpu.VMEM((CH, w), jnp.float32),
                     pltpu.VMEM((CH, w), jnp.float32),
                     pltpu.VMEM((CH, w), jnp.float32),
                     pltpu.VMEM((CH, w), jnp.float32),
                     pltpu.VMEM((8,), jnp.float32),
                     pltpu.SemaphoreType.DMA,
                     pltpu.SemaphoreType.DMA,
                     pltpu.SemaphoreType.DMA],
      compiler_params=_SC_PARAMS)
  def gather_k(t_h, src_h, dst_h, tok_h, hI_h, hJ_h, pI_h, pJ_h, tok_o,
               idxd, idxs, bg0, bf0, bg1, bf1, tokb, sem0, sem1, semw):
    wid = lax.axis_index("s") * NC + lax.axis_index("c")
    base0 = wid * EPW

    @pl.when(wid == 0)
    def _():
      pltpu.sync_copy(tok_h, tokb)
      pltpu.sync_copy(tokb, tok_o)
    pltpu.sync_copy(dst_h.at[pl.ds(base0, EPW)], idxd)
    pltpu.sync_copy(src_h.at[pl.ds(base0, EPW)], idxs)
    npairs = EPW // (2 * CH)

    def fire(off, bg, bf, sem):
      cg = pltpu.async_copy(t_h.at[idxd.at[pl.ds(off, CH)]], bg, sem)
      cf = pltpu.async_copy(t_h.at[idxs.at[pl.ds(off, CH)]], bf, sem)
      return cg, cf

    def drain(off, bg, bf, cg, cf):
      cg.wait()
      cf.wait()
      sl = pl.ds(base0 + off, CH)
      wg = pltpu.async_copy(bg.at[:, pl.ds(0, e)], hI_h.at[sl], semw)
      wf = pltpu.async_copy(bf.at[:, pl.ds(0, e)], hJ_h.at[sl], semw)
      wp = pltpu.async_copy(bg.at[:, pl.ds(e, 8)], pI_h.at[sl], semw)
      wq = pltpu.async_copy(bf.at[:, pl.ds(e, 8)], pJ_h.at[sl], semw)
      return wg, wf, wp, wq

    def body(pi, carry):
      off0 = 2 * pi * CH
      off1 = off0 + CH
      cg0, cf0 = fire(off0, bg0, bf0, sem0)
      cg1, cf1 = fire(off1, bg1, bf1, sem1)
      ws0 = drain(off0, bg0, bf0, cg0, cf0)
      ws1 = drain(off1, bg1, bf1, cg1, cf1)
      for c in ws0 + ws1:
        c.wait()
      return carry

    lax.fori_loop(0, npairs, body, 0)
    for off in range(2 * npairs * CH, EPW, CH):
      cg0, cf0 = fire(off, bg0, bf0, sem0)
      for c in drain(off, bg0, bf0, cg0, cf0):
        c.wait()

  return gather_k


@functools.cache
def _make_scatter(e, with_msg):
  outs = []
  if with_msg:
    outs.append(jax.ShapeDtypeStruct((NC, N, e), jnp.float32))
  outs.append(jax.ShapeDtypeStruct((NC, N, 16), jnp.float32))
  outs.append(jax.ShapeDtypeStruct((8,), jnp.float32))
  scratch = [pltpu.VMEM((8,), jnp.float32),
             pltpu.VMEM((CH,), jnp.int32), pltpu.VMEM((CH,), jnp.int32)]
  if with_msg:
    scratch += [pltpu.VMEM((CH, e), jnp.float32),
                pltpu.VMEM((CH, e), jnp.float32)]
  scratch += [pltpu.VMEM((CH, 16), jnp.float32),
              pltpu.VMEM((CH, 16), jnp.float32)]
  if with_msg:
    scratch.append(pltpu.VMEM_SHARED((N, e), jnp.float32))
  scratch += [pltpu.VMEM_SHARED((N, 16), jnp.float32),
              pltpu.SemaphoreType.DMA, pltpu.SemaphoreType.DMA]

  @functools.partial(pl.kernel, out_type=tuple(outs), mesh=_mesh(),
                     scratch_types=scratch, compiler_params=_SC_PARAMS)
  def scatter_k(*refs):
    if with_msg:
      (msgP_h, posP_h, dst_h, zm_h, zp_h, tok_h, accM_h, accP_h, tok_o,
       tokb, bi0, bi1, bm0, bm1, bp0, bp1, spM, spP, sem0, sem1) = refs
    else:
      (posP_h, dst_h, zp_h, tok_h, accP_h, tok_o,
       tokb, bi0, bi1, bp0, bp1, spP, sem0, sem1) = refs
      msgP_h = spM = bm0 = bm1 = None
    cid = lax.axis_index("c")
    sid = lax.axis_index("s")
    wid = sid * NC + cid
    rbase = sid * RPT

    @pl.when(wid == 0)
    def _():
      pltpu.sync_copy(tok_h, tokb)
      pltpu.sync_copy(tokb, tok_o)
    pltpu.sync_copy(zp_h, spP.at[pl.ds(rbase, RPT)])
    if with_msg:
      pltpu.sync_copy(zm_h, spM.at[pl.ds(rbase, RPT)])
    plsc.subcore_barrier()
    base0 = wid * EPW
    npairs = EPW // (2 * CH)

    def fire(off, bi, bm, bp, sem):
      sl = pl.ds(base0 + off, CH)
      cs = [pltpu.async_copy(dst_h.at[sl], bi, sem),
            pltpu.async_copy(posP_h.at[sl], bp, sem)]
      if with_msg:
        cs.append(pltpu.async_copy(msgP_h.at[sl], bm, sem))
      return cs

    def scat(bi, bm, bp, cs):
      for c in cs:
        c.wait()
      pltpu.sync_copy(bp, spP.at[bi], add=True)
      if with_msg:
        pltpu.sync_copy(bm, spM.at[bi], add=True)

    def body(pi, carry):
      off0 = 2 * pi * CH
      c0 = fire(off0, bi0, bm0, bp0, sem0)
      c1 = fire(off0 + CH, bi1, bm1, bp1, sem1)
      scat(bi0, bm0, bp0, c0)
      scat(bi1, bm1, bp1, c1)
      return carry

    lax.fori_loop(0, npairs, body, 0)
    for off in range(2 * npairs * CH, EPW, CH):
      c0 = fire(off, bi0, bm0, bp0, sem0)
      scat(bi0, bm0, bp0, c0)
    plsc.subcore_barrier()
    pltpu.sync_copy(spP.at[pl.ds(rbase, RPT)],
                    accP_h.at[cid, pl.ds(rbase, RPT)])
    if with_msg:
      pltpu.sync_copy(spM.at[pl.ds(rbase, RPT)],
                      accM_h.at[cid, pl.ds(rbase, RPT)])

  return scatter_k


def _gather_edges(hx, src, dst, w, tok):
  hI, hJ, pI, pJ, tok2 = _make_gather(w)(hx, src, dst, tok)
  return (hI, hJ, pI, pJ), tok2


def _scatter_edges(msgP, posP, dst, e, tok):
  zp = jnp.zeros((RPT, 16), jnp.float32)
  if msgP is not None:
    zm = jnp.zeros((RPT, e), jnp.float32)
    accM, accP, tok2 = _make_scatter(e, True)(msgP, posP, dst, zm, zp, tok)
    return accM, accP, tok2
  accP, tok2 = _make_scatter(e, False)(posP, dst, zp, tok)
  return None, accP, tok2


def _bn_affine(st, g, b, denom):
  m = st[0:1, :]
  v = st[1:2, :] / denom
  scale = g * lax.rsqrt(v + EPS)
  return scale, b - m * scale


def _acc_stats(st_ref, z, nb):
  m_b = jnp.sum(z, 0, keepdims=True) / float(nb)
  zc = z - m_b
  ssw = jnp.sum(zc * zc, 0, keepdims=True)
  i = pl.program_id(0)

  @pl.when(i == 0)
  def _():
    st_ref[...] = jnp.concatenate([m_b, ssw], 0)

  @pl.when(i > 0)
  def _():
    n_prev = (i * nb).astype(jnp.float32)
    n_new = n_prev + float(nb)
    mean_prev = st_ref[0:1, :]
    delta = m_b - mean_prev
    st_ref[0:1, :] = mean_prev + delta * (float(nb) / n_new)
    st_ref[1:2, :] += ssw + delta * delta * (n_prev * float(nb) / n_new)


def _z1_block(W1_ref, b1_ref, hI_ref, hJ_ref, pI_ref, pJ_ref):
  d = pI_ref[...] - pJ_ref[...]
  dist = jnp.sqrt(jnp.sum(d[:, 0:3] * d[:, 0:3], axis=1, keepdims=True))
  msg = jnp.concatenate([hI_ref[...], hJ_ref[...], dist], axis=1)
  return (jnp.dot(msg, W1_ref[...], preferred_element_type=jnp.float32)
          + b1_ref[...]), d


def _eblk(e):
  return pl.BlockSpec((BE, e), lambda i: (i, 0))


def _cblk(r, e):
  return pl.BlockSpec((r, e), lambda i: (0, 0))


@functools.cache
def _make_k1s(e):
  def body(W1, b1, hI, hJ, pI, pJ, z1_out, pd_out, st):
    z1, d = _z1_block(W1, b1, hI, hJ, pI, pJ)
    z1_out[...] = z1
    pd_out[...] = d
    _acc_stats(st, z1, BE)

  return pl.pallas_call(
      body, grid=(GRID,),
      in_specs=[_cblk(2 * e + 1, e), _cblk(1, e),
                _eblk(e), _eblk(e), _eblk(8), _eblk(8)],
      out_specs=(_eblk(e), _eblk(8), _cblk(2, e)),
      out_shape=(jax.ShapeDtypeStruct((E, e), jnp.float32),
                 jax.ShapeDtypeStruct((E, 8), jnp.float32),
                 jax.ShapeDtypeStruct((2, e), jnp.float32)))


@functools.cache
def _make_k2(e):
  def body(st1, g1, bb1, W2, b2, z1, z2, st2):
    scale, shift = _bn_affine(st1[...], g1[...], bb1[...], float(E))
    a1 = jnp.maximum(z1[...] * scale + shift, 0.0)
    z2v = jnp.dot(a1, W2[...], preferred_element_type=jnp.float32) + b2[...]
    z2[...] = z2v
    _acc_stats(st2, z2v, BE)

  return pl.pallas_call(
      body, grid=(GRID,),
      in_specs=[_cblk(2, e), _cblk(1, e), _cblk(1, e),
                _cblk(e, e), _cblk(1, e), _eblk(e)],
      out_specs=(_eblk(e), _cblk(2, e)),
      out_shape=(jax.ShapeDtypeStruct((E, e), jnp.float32),
                 jax.ShapeDtypeStruct((2, e), jnp.float32)))


@functools.cache
def _make_k3(e):
  def body(st2, g2, b2, Wp1, bp1, z2, z3, st3):
    scale, shift = _bn_affine(st2[...], g2[...], b2[...], float(E))
    msg = jnp.maximum(z2[...] * scale + shift, 0.0)
    z3v = jnp.dot(msg, Wp1[...], preferred_element_type=jnp.float32) + bp1[...]
    z3[...] = z3v
    _acc_stats(st3, z3v, BE)

  return pl.pallas_call(
      body, grid=(GRID,),
      in_specs=[_cblk(2, e), _cblk(1, e), _cblk(1, e),
                _cblk(e, e), _cblk(1, e), _eblk(e)],
      out_specs=(_eblk(e), _cblk(2, e)),
      out_shape=(jax.ShapeDtypeStruct((E, e), jnp.float32),
                 jax.ShapeDtypeStruct((2, e), jnp.float32)))


@functools.cache
def _make_k4(e, with_msg):
  def body(*refs):
    if with_msg:
      (st2, g2, b2, st3, g3, b3, wp2, bp2, z2, z3, pd, *pay) = refs
    else:
      (st3, g3, b3, wp2, bp2, z3, pd, pay) = refs
    scale3, shift3 = _bn_affine(st3[...], g3[...], b3[...], float(E))
    a3 = jnp.maximum(z3[...] * scale3 + shift3, 0.0)
    w = jnp.dot(a3, wp2[...], preferred_element_type=jnp.float32) + bp2[...]
    wpos = pd[...][:, 0:3] * w
    tail = jnp.concatenate(
        [wpos, jnp.ones((BE, 1), jnp.float32),
         jnp.zeros((BE, 12), jnp.float32)], axis=1)
    if with_msg:
      scale2, shift2 = _bn_affine(st2[...], g2[...], b2[...], float(E))
      msg = jnp.maximum(z2[...] * scale2 + shift2, 0.0)
      msgP, posP = pay
      msgP[...] = msg
      posP[...] = tail
    else:
      pay[...] = tail

  if with_msg:
    in_specs = [_cblk(2, e), _cblk(1, e), _cblk(1, e),
                _cblk(2, e), _cblk(1, e), _cblk(1, e),
                _cblk(e, 1), _cblk(1, 1),
                _eblk(e), _eblk(e), _eblk(8)]
    out_specs = (_eblk(e), _eblk(16))
    out_shape = (jax.ShapeDtypeStruct((E, e), jnp.float32),
                 jax.ShapeDtypeStruct((E, 16), jnp.float32))
  else:
    in_specs = [_cblk(2, e), _cblk(1, e), _cblk(1, e),
                _cblk(e, 1), _cblk(1, 1),
                _eblk(e), _eblk(8)]
    out_specs = _eblk(16)
    out_shape = jax.ShapeDtypeStruct((E, 16), jnp.float32)

  return pl.pallas_call(body, grid=(GRID,), in_specs=in_specs,
                        out_specs=out_specs, out_shape=out_shape)


BNODE = 2000
NGRID = N // BNODE


def _nblk(e):
  return pl.BlockSpec((BNODE, e), lambda i: (i, 0))


def _nblk3(e):
  return pl.BlockSpec((NC, BNODE, e), lambda i: (0, i, 0))


def _ncblk(r, e):
  return pl.BlockSpec((r, e), lambda i: (0, 0))


def _dot(a, b):
  return jnp.dot(a, b, preferred_element_type=jnp.float32)


@functools.cache
def _make_node_a(e):
  def body(h, pos4, accM, accP, Wu1, bu1, z, pos4_out, st):
    aPc = accP[...]
    aP = aPc[0] + aPc[1]
    cnt = jnp.maximum(aP[:, 3:4], 1.0)
    aMc = accM[...]
    aM = (aMc[0] + aMc[1]) / cnt
    zv = _dot(jnp.concatenate([h[...], aM], axis=1), Wu1[...]) + bu1[...]
    z[...] = zv
    pos4_out[...] = pos4[...] + jnp.concatenate(
        [aP[:, 0:3] / cnt, jnp.zeros((BNODE, 5), jnp.float32)], axis=1)
    _acc_stats(st, zv, BNODE)

  return pl.pallas_call(
      body, grid=(NGRID,),
      in_specs=[_nblk(e), _nblk(8), _nblk3(e), _nblk3(16),
                _ncblk(2 * e, e), _ncblk(1, e)],
      out_specs=(_nblk(e), _nblk(8), _ncblk(2, e)),
      out_shape=(jax.ShapeDtypeStruct((N, e), jnp.float32),
                 jax.ShapeDtypeStruct((N, 8), jnp.float32),
                 jax.ShapeDtypeStruct((2, e), jnp.float32)))


@functools.cache
def _make_node_b(e):
  def body(st1, g1, b1, Wu2, bu2, z, z2, st2):
    scale, shift = _bn_affine(st1[...], g1[...], b1[...], float(N))
    u = jnp.maximum(z[...] * scale + shift, 0.0)
    z2v = _dot(u, Wu2[...]) + bu2[...]
    z2[...] = z2v
    _acc_stats(st2, z2v, BNODE)

  return pl.pallas_call(
      body, grid=(NGRID,),
      in_specs=[_ncblk(2, e), _ncblk(1, e), _ncblk(1, e),
                _ncblk(e, e), _ncblk(1, e), _nblk(e)],
      out_specs=(_nblk(e), _ncblk(2, e)),
      out_shape=(jax.ShapeDtypeStruct((N, e), jnp.float32),
                 jax.ShapeDtypeStruct((2, e), jnp.float32)))


@functools.cache
def _make_node_c(e, e_out):
  def body(st2, g2, b2, Wo, bo, z2, h_out):
    scale, shift = _bn_affine(st2[...], g2[...], b2[...], float(N))
    u2 = jnp.maximum(z2[...] * scale + shift, 0.0)
    h_out[...] = _dot(u2, Wo[...]) + bo[...]

  return pl.pallas_call(
      body, grid=(NGRID,),
      in_specs=[_ncblk(2, e), _ncblk(1, e), _ncblk(1, e),
                _ncblk(e, e_out), _ncblk(1, e_out), _nblk(e)],
      out_specs=_nblk(e_out),
      out_shape=jax.ShapeDtypeStruct((N, e_out), jnp.float32))


@functools.cache
def _make_posnode():
  def body(pos4, accP, pos4_out):
    aPc = accP[...]
    aP = aPc[0] + aPc[1]
    cnt = jnp.maximum(aP[:, 3:4], 1.0)
    pos4_out[...] = pos4[...] + jnp.concatenate(
        [aP[:, 0:3] / cnt, jnp.zeros((N, 5), jnp.float32)], axis=1)

  return pl.pallas_call(
      body, out_shape=jax.ShapeDtypeStruct((N, 8), jnp.float32))


@functools.cache
def _make_init(e0):
  def body(x, Wi, bi, h0):
    h0[...] = x[...] * Wi[...] + bi[...]

  return pl.pallas_call(
      body, grid=(NGRID,),
      in_specs=[_nblk(1), _ncblk(1, 64), _ncblk(1, 64)],
      out_specs=_nblk(64),
      out_shape=jax.ShapeDtypeStruct((N, 64), jnp.float32))


_DIMS = [64, 128, 256]


def _row(a):
  return a.reshape(1, -1)


def _run_pair(gs):
  for g in gs:
    g["pos4"] = jnp.pad(g["pos"], ((0, 0), (0, 5)))
    g["src"] = g["ei"][0]
    g["dst"] = g["ei"][1]
    g["h"] = _make_init(64)(g["x"], g["lin"]["W"], _row(g["lin"]["b"]))
  gs[0]["tok"] = jnp.zeros((8,), jnp.float32)
  nl = len(gs[0]["layers"])
  for li in range(nl):
    e = _DIMS[li]
    last = li == nl - 1
    for g in gs:
      hx = jnp.concatenate([g["h"], g["pos4"]], axis=1)
      g["gath"], gs[0]["tok"] = _gather_edges(
          hx, g["src"], g["dst"], e + 8, gs[0]["tok"])
    for g in gs:
      lp = g["layers"][li]
      hI, hJ, pI, pJ = g["gath"]
      g["z1"], g["pd"], g["st1"] = _make_k1s(e)(
          lp["msg1"]["W"], _row(lp["msg1"]["b"]), hI, hJ, pI, pJ)
    for g in gs:
      lp = g["layers"][li]
      g["z2"], g["st2"] = _make_k2(e)(
          g["st1"], _row(lp["msg_bn1"]["g"]), _row(lp["msg_bn1"]["b"]),
          lp["msg2"]["W"], _row(lp["msg2"]["b"]), g["z1"])
    for g in gs:
      lp = g["layers"][li]
      g["z3"], g["st3"] = _make_k3(e)(
          g["st2"], _row(lp["msg_bn2"]["g"]), _row(lp["msg_bn2"]["b"]),
          lp["pos1"]["W"], _row(lp["pos1"]["b"]), g["z2"])
    for g in gs:
      lp = g["layers"][li]
      bn2g, bn2b = _row(lp["msg_bn2"]["g"]), _row(lp["msg_bn2"]["b"])
      bnpg, bnpb = _row(lp["pos_bn"]["g"]), _row(lp["pos_bn"]["b"])
      wp2 = lp["pos2"]["W"]
      bp2 = lp["pos2"]["b"].reshape(1, 1)
      if not last:
        g["msgP"], g["posP"] = _make_k4(e, True)(
            g["st2"], bn2g, bn2b, g["st3"], bnpg, bnpb,
            wp2, bp2, g["z2"], g["z3"], g["pd"])
      else:
        g["msgP"] = None
        g["posP"] = _make_k4(e, False)(
            g["st3"], bnpg, bnpb, wp2, bp2, g["z3"], g["pd"])
    for g in gs:
      g["accM"], g["accP"], gs[0]["tok"] = _scatter_edges(
          g["msgP"], g["posP"], g["dst"], e, gs[0]["tok"])
    if not last:
      e_out = _DIMS[li + 1]
      for g in gs:
        lp = g["layers"][li]
        g["z"], g["pos4"], g["stu1"] = _make_node_a(e)(
            g["h"], g["pos4"], g["accM"], g["accP"],
            lp["upd1"]["W"], _row(lp["upd1"]["b"]))
      for g in gs:
        lp = g["layers"][li]
        g["z2n"], g["stu2"] = _make_node_b(e)(
            g["stu1"], _row(lp["upd_bn1"]["g"]), _row(lp["upd_bn1"]["b"]),
            lp["upd2"]["W"], _row(lp["upd2"]["b"]), g["z"])
      for g in gs:
        lp = g["layers"][li]
        g["h"] = _make_node_c(e, e_out)(
            g["stu2"], _row(lp["upd_bn2"]["g"]), _row(lp["upd_bn2"]["b"]),
            lp["out"]["W"], _row(lp["out"]["b"]), g["z2n"])
    else:
      for g in gs:
        g["pos4"] = _make_posnode()(g["pos4"], g["accP"])
  return tuple(g["pos4"][:, :3] for g in gs)


def kernel(rec_x, rec_pos, rec_edge_index, lig_x, lig_pos, lig_edge_index,
           params):
  gs = [
      dict(x=rec_x, pos=rec_pos, ei=rec_edge_index,
           lin=params["lin_in_rec"], layers=params["rec_layers"]),
      dict(x=lig_x, pos=lig_pos, ei=lig_edge_index,
           lin=params["lin_in_lig"], layers=params["lig_layers"]),
  ]
  p_r, p_l = _run_pair(gs)
  return (p_r, p_l)
